# baseline (speedup 1.0000x reference)
import jax
import jax.numpy as jnp
from jax import lax
from jax.experimental import pallas as pl
from jax.experimental.pallas import tpu as pltpu


def _fused_kernel(x_ref, w_ref, b_ref, o_ref):
    x = x_ref[...]
    y = lax.dot_general(
        x,
        w_ref[...],
        dimension_numbers=(((1,), (1,)), ((), ())),
        preferred_element_type=jnp.float32,
    )
    o_ref[...] = x + y + b_ref[...]


def kernel(x2d, w_out_in, b):
    M, H = x2d.shape
    TM = 512
    m_pad = pl.cdiv(M, TM) * TM
    x_in = x2d if m_pad == M else jnp.pad(x2d, ((0, m_pad - M), (0, 0)))
    m_tiles = m_pad // TM

    out = pl.pallas_call(
        _fused_kernel,
        out_shape=jax.ShapeDtypeStruct((TM, H), x2d.dtype),
        grid=(m_tiles,),
        in_specs=[
            pl.BlockSpec((TM, H), lambda i: (0, 0)),
            pl.BlockSpec((H, H), lambda i: (0, 0)),
            pl.BlockSpec((1, H), lambda i: (0, 0)),
        ],
        out_specs=pl.BlockSpec((TM, H), lambda i: (0, 0)),
        compiler_params=pltpu.CompilerParams(
            dimension_semantics=("arbitrary",),
            vmem_limit_bytes=60 * 1024 * 1024,
        ),
        cost_estimate=pl.CostEstimate(
            flops=2 * m_pad * H * H,
            transcendentals=0,
            bytes_accessed=2 * m_pad * H * 4 + w_out_in.nbytes + b.nbytes,
        ),
    )(x_in, w_out_in, b.reshape(1, H))

    return jnp.broadcast_to(out[:1], (M, H))


# speedup vs baseline: 1.1853x; 1.1853x over previous
import jax
import jax.numpy as jnp
from jax import lax
from jax.experimental import pallas as pl
from jax.experimental.pallas import tpu as pltpu

_TM = 1024
_TN = 1024


def _fused_kernel(x_ref, w_ref, b_ref, o_ref):
    j = pl.program_id(1)
    w = w_ref[pl.ds(j * _TN, _TN), :]
    y = lax.dot_general(
        x_ref[...],
        w,
        dimension_numbers=(((1,), (1,)), ((), ())),
        preferred_element_type=jnp.float32,
    )
    xres = x_ref[:, pl.ds(j * _TN, _TN)]
    o_ref[...] = xres + y + b_ref[:, pl.ds(j * _TN, _TN)]


def kernel(x2d, w_out_in, b):
    M, H = x2d.shape
    m_pad = pl.cdiv(M, _TM) * _TM
    x_in = x2d if m_pad == M else jnp.pad(x2d, ((0, m_pad - M), (0, 0)))
    m_tiles = m_pad // _TM
    n_tiles = H // _TN

    out = pl.pallas_call(
        _fused_kernel,
        out_shape=jax.ShapeDtypeStruct((m_pad, H), x2d.dtype),
        grid=(m_tiles, n_tiles),
        in_specs=[
            pl.BlockSpec((_TM, H), lambda i, j: (i, 0)),
            pl.BlockSpec((H, H), lambda i, j: (0, 0)),
            pl.BlockSpec((1, H), lambda i, j: (0, 0)),
        ],
        out_specs=pl.BlockSpec((_TM, _TN), lambda i, j: (i, j)),
        compiler_params=pltpu.CompilerParams(
            dimension_semantics=("arbitrary", "arbitrary"),
            vmem_limit_bytes=60 * 1024 * 1024,
        ),
        cost_estimate=pl.CostEstimate(
            flops=2 * m_pad * H * H,
            transcendentals=0,
            bytes_accessed=2 * m_pad * H * 4 + w_out_in.nbytes + b.nbytes,
        ),
    )(x_in, w_out_in, b.reshape(1, H))

    return out[:M] if m_pad != M else out
